# baseline (speedup 1.0000x reference)
import functools

import jax
import jax.numpy as jnp
from jax import lax
from jax.experimental import pallas as pl
from jax.experimental.pallas import tpu as pltpu
from jax.experimental.pallas import tpu_sc as plsc

N_ROWS = 320000
D_FEAT = 128
NUM_SEGMENTS = 10000

NC = 2
NS = 16
HALF = NUM_SEGMENTS // 2
DUMMY = HALF
ACC_ROWS = HALF + 16
BLK = 256
SCAT = 128
ZROWS = 24
ZSTRIPE = 312
ZLAST = ACC_ROWS - (NS - 1) * ZSTRIPE
WSTRIPE = 312
WLAST = HALF - (NS - 1) * WSTRIPE


def _sc_segment_sum(X, keysl2, r8arr):
    mesh = plsc.VectorSubcoreMesh(core_axis_name="c", subcore_axis_name="s")

    @functools.partial(
        pl.kernel,
        out_type=jax.ShapeDtypeStruct((NUM_SEGMENTS, D_FEAT), jnp.float32),
        mesh=mesh,
        scratch_types=[
            pltpu.VMEM((BLK, D_FEAT), jnp.float32),
            pltpu.VMEM((BLK, D_FEAT), jnp.float32),
            pltpu.VMEM((SCAT,), jnp.int32),
            pltpu.VMEM((SCAT,), jnp.int32),
            pltpu.VMEM((SCAT,), jnp.int32),
            pltpu.VMEM((SCAT,), jnp.int32),
            pltpu.VMEM((16, D_FEAT), jnp.float32),
            pltpu.VMEM((16,), jnp.int32),
            pltpu.VMEM((16,), jnp.int32),
            pltpu.VMEM((ZROWS, D_FEAT), jnp.float32),
            pltpu.VMEM_SHARED((ACC_ROWS, D_FEAT), jnp.float32),
            pltpu.SemaphoreType.DMA,
            pltpu.SemaphoreType.DMA,
            pltpu.SemaphoreType.DMA,
        ],
    )
    def k(x_hbm, keys_hbm, r8_hbm, out_hbm,
          xbuf_a, xbuf_b, klo_a, khi_a, klo_b, khi_b,
          sbuf, skbuf, rbuf, zbuf, acc, sem_a, sem_b, sem_z):
        c = lax.axis_index("c")
        s = lax.axis_index("s")

        pltpu.sync_copy(r8_hbm, rbuf)
        r8v = pl.multiple_of(rbuf[...][0], 8)

        kb0 = pl.multiple_of(c * N_ROWS, 8)
        start_c = jnp.where(c == 0, 0, r8v + 16)
        end_c = jnp.where(c == 0, r8v, N_ROWS)
        count_c = end_c - start_c
        per = (count_c // (NS * BLK)) * BLK
        base_s = pl.multiple_of(start_c + s * per, 8)
        count_last = count_c - (NS - 1) * per
        nfull = jnp.where(s < NS - 1, per // BLK, count_last // BLK)
        rem = count_last - (count_last // BLK) * BLK

        def start_load(i, xbuf, klo, khi, sem):
            boff = pl.multiple_of(base_s + i * BLK, 8)
            pltpu.async_copy(x_hbm.at[pl.ds(boff, BLK)], xbuf, sem)
            pltpu.async_copy(keys_hbm.at[pl.ds(kb0 + boff, SCAT)], klo, sem)
            pltpu.async_copy(
                keys_hbm.at[pl.ds(kb0 + boff + SCAT, SCAT)], khi, sem)

        def wait_load(i, xbuf, klo, khi, sem):
            boff = pl.multiple_of(base_s + i * BLK, 8)
            pltpu.make_async_copy(
                x_hbm.at[pl.ds(boff, BLK)], xbuf, sem).wait()
            pltpu.make_async_copy(
                keys_hbm.at[pl.ds(kb0 + boff, SCAT)], klo, sem).wait()
            pltpu.make_async_copy(
                keys_hbm.at[pl.ds(kb0 + boff + SCAT, SCAT)], khi, sem).wait()

        def scatter(xbuf, klo, khi):
            pltpu.sync_copy(xbuf.at[pl.ds(0, SCAT)], acc.at[klo], add=True)
            pltpu.sync_copy(xbuf.at[pl.ds(SCAT, SCAT)], acc.at[khi],
                            add=True)

        @pl.when(nfull > 0)
        def _():
            start_load(0, xbuf_a, klo_a, khi_a, sem_a)

        @pl.when(nfull > 1)
        def _():
            start_load(1, xbuf_b, klo_b, khi_b, sem_b)

        @pl.loop(0, ZROWS)
        def _(r):
            @pl.loop(0, D_FEAT, step=16)
            def _(col):
                zbuf[r, pl.ds(col, 16)] = jnp.zeros((16,), jnp.float32)

        zbase = pl.multiple_of(s * ZSTRIPE, 8)
        nz = jnp.where(s < NS - 1, ZSTRIPE // ZROWS, ZLAST // ZROWS)

        @pl.loop(0, ZLAST // ZROWS)
        def _(j):
            @pl.when(j < nz)
            def _():
                pltpu.async_copy(
                    zbuf, acc.at[pl.ds(zbase + j * ZROWS, ZROWS)], sem_z)

        @pl.loop(0, ZLAST // ZROWS)
        def _(j):
            @pl.when(j < nz)
            def _():
                pltpu.make_async_copy(
                    zbuf, acc.at[pl.ds(zbase + j * ZROWS, ZROWS)],
                    sem_z).wait()

        plsc.subcore_barrier()

        lanes = lax.iota(jnp.int32, 16)

        @pl.when(s == 0)
        def _():
            pltpu.sync_copy(x_hbm.at[pl.ds(r8v, 16)], sbuf)
            pltpu.sync_copy(keys_hbm.at[pl.ds(kb0 + r8v, 16)], skbuf)
            v = skbuf[...]
            valid = (v >= 0) & (v < HALF)
            skbuf[...] = jnp.where(valid, v, DUMMY + lanes)
            pltpu.sync_copy(sbuf, acc.at[skbuf], add=True)

        @pl.loop(0, (nfull + 1) // 2)
        def _(g):
            i0 = 2 * g
            wait_load(i0, xbuf_a, klo_a, khi_a, sem_a)
            scatter(xbuf_a, klo_a, khi_a)

            @pl.when(i0 + 2 < nfull)
            def _():
                start_load(i0 + 2, xbuf_a, klo_a, khi_a, sem_a)

            @pl.when(i0 + 1 < nfull)
            def _():
                wait_load(i0 + 1, xbuf_b, klo_b, khi_b, sem_b)
                scatter(xbuf_b, klo_b, khi_b)

                @pl.when(i0 + 3 < nfull)
                def _():
                    start_load(i0 + 3, xbuf_b, klo_b, khi_b, sem_b)

        @pl.when((s == NS - 1) & (rem > 0))
        def _():
            blockstart = pl.multiple_of(jnp.maximum(end_c - BLK, 0), 8)
            lo = base_s + (count_last // BLK) * BLK
            pltpu.sync_copy(x_hbm.at[pl.ds(blockstart, BLK)], xbuf_a)
            pltpu.sync_copy(keys_hbm.at[pl.ds(kb0 + blockstart, SCAT)], klo_a)
            pltpu.sync_copy(
                keys_hbm.at[pl.ds(kb0 + blockstart + SCAT, SCAT)], khi_a)
            for half_i, kb in ((0, klo_a), (1, khi_a)):
                for q in range(SCAT // 16):
                    pos = blockstart + half_i * SCAT + q * 16 + lanes
                    v = kb[pl.ds(q * 16, 16)]
                    valid = (pos >= lo) & (pos < end_c)
                    kb[pl.ds(q * 16, 16)] = jnp.where(
                        valid, v, DUMMY + lanes)
            scatter(xbuf_a, klo_a, khi_a)

        plsc.subcore_barrier()

        obase = pl.multiple_of(c * HALF + s * WSTRIPE, 8)

        @pl.when(s < NS - 1)
        def _():
            pltpu.sync_copy(
                acc.at[pl.ds(s * WSTRIPE, WSTRIPE)],
                out_hbm.at[pl.ds(obase, WSTRIPE)],
            )

        @pl.when(s == NS - 1)
        def _():
            pltpu.sync_copy(
                acc.at[pl.ds((NS - 1) * WSTRIPE, WLAST)],
                out_hbm.at[pl.ds(obase, WLAST)],
            )

    return k(X, keysl2, r8arr)


@jax.jit
def kernel(X, keys):
    keys32 = keys.astype(jnp.int32)
    r_star = jnp.searchsorted(keys32, jnp.int32(HALF)).astype(jnp.int32)
    r8 = jnp.minimum((r_star // 8) * 8, N_ROWS - 16)
    r8arr = jnp.full((16,), r8, jnp.int32)
    keysl2 = jnp.concatenate([keys32, keys32 - HALF])
    return _sc_segment_sum(X, keysl2, r8arr)


# speedup vs baseline: 1.1563x; 1.1563x over previous
import functools

import jax
import jax.numpy as jnp
from jax import lax
from jax.experimental import pallas as pl
from jax.experimental.pallas import tpu as pltpu
from jax.experimental.pallas import tpu_sc as plsc

N_ROWS = 320000
D_FEAT = 128
NUM_SEGMENTS = 10000

NC = 2
NS = 16
HALF = NUM_SEGMENTS // 2
DUMMY = HALF
ACC_ROWS = HALF + 16
BLK = 64
NBUF = 4
ZROWS = 24
ZSTRIPE = 312
ZLAST = ACC_ROWS - (NS - 1) * ZSTRIPE
WSTRIPE = 312
WLAST = HALF - (NS - 1) * WSTRIPE


def _sc_segment_sum(X, keysl2, r8arr):
    mesh = plsc.VectorSubcoreMesh(core_axis_name="c", subcore_axis_name="s")

    @functools.partial(
        pl.kernel,
        out_type=jax.ShapeDtypeStruct((NUM_SEGMENTS, D_FEAT), jnp.float32),
        mesh=mesh,
        scratch_types=[
            pltpu.VMEM((BLK, D_FEAT), jnp.float32),
            pltpu.VMEM((BLK, D_FEAT), jnp.float32),
            pltpu.VMEM((BLK, D_FEAT), jnp.float32),
            pltpu.VMEM((BLK, D_FEAT), jnp.float32),
            pltpu.VMEM((BLK,), jnp.int32),
            pltpu.VMEM((BLK,), jnp.int32),
            pltpu.VMEM((BLK,), jnp.int32),
            pltpu.VMEM((BLK,), jnp.int32),
            pltpu.VMEM((16, D_FEAT), jnp.float32),
            pltpu.VMEM((16,), jnp.int32),
            pltpu.VMEM((16,), jnp.int32),
            pltpu.VMEM((ZROWS, D_FEAT), jnp.float32),
            pltpu.VMEM_SHARED((ACC_ROWS, D_FEAT), jnp.float32),
            pltpu.SemaphoreType.DMA,
            pltpu.SemaphoreType.DMA,
            pltpu.SemaphoreType.DMA,
            pltpu.SemaphoreType.DMA,
            pltpu.SemaphoreType.DMA,
        ],
    )
    def k(x_hbm, keys_hbm, r8_hbm, out_hbm,
          xbuf_a, xbuf_b, xbuf_c, xbuf_d, kb_a, kb_b, kb_c, kb_d,
          sbuf, skbuf, rbuf, zbuf, acc,
          sem_a, sem_b, sem_c, sem_d, sem_z):
        c = lax.axis_index("c")
        s = lax.axis_index("s")

        pltpu.sync_copy(r8_hbm, rbuf)
        r8v = pl.multiple_of(rbuf[...][0], 8)

        kb0 = pl.multiple_of(c * N_ROWS, 8)
        start_c = jnp.where(c == 0, 0, r8v + 16)
        end_c = jnp.where(c == 0, r8v, N_ROWS)
        count_c = end_c - start_c
        per = (count_c // (NS * BLK)) * BLK
        base_s = pl.multiple_of(start_c + s * per, 8)
        count_last = count_c - (NS - 1) * per
        nfull = jnp.where(s < NS - 1, per // BLK, count_last // BLK)
        rem = count_last - (count_last // BLK) * BLK

        def start_load(i, xbuf, kbuf, sem):
            boff = pl.multiple_of(base_s + i * BLK, 8)
            pltpu.async_copy(x_hbm.at[pl.ds(boff, BLK)], xbuf, sem)
            pltpu.async_copy(keys_hbm.at[pl.ds(kb0 + boff, BLK)], kbuf, sem)

        def wait_load(i, xbuf, kbuf, sem):
            boff = pl.multiple_of(base_s + i * BLK, 8)
            pltpu.make_async_copy(
                x_hbm.at[pl.ds(boff, BLK)], xbuf, sem).wait()
            pltpu.make_async_copy(
                keys_hbm.at[pl.ds(kb0 + boff, BLK)], kbuf, sem).wait()

        def scatter(xbuf, kbuf):
            pltpu.sync_copy(xbuf, acc.at[kbuf], add=True)

        bufs = ((xbuf_a, kb_a, sem_a), (xbuf_b, kb_b, sem_b),
                (xbuf_c, kb_c, sem_c), (xbuf_d, kb_d, sem_d))

        for b in range(NBUF):
            @pl.when(nfull > b)
            def _(b=b):
                start_load(b, *bufs[b])

        @pl.loop(0, ZROWS)
        def _(r):
            @pl.loop(0, D_FEAT, step=16)
            def _(col):
                zbuf[r, pl.ds(col, 16)] = jnp.zeros((16,), jnp.float32)

        zbase = pl.multiple_of(s * ZSTRIPE, 8)
        nz = jnp.where(s < NS - 1, ZSTRIPE // ZROWS, ZLAST // ZROWS)

        @pl.loop(0, ZLAST // ZROWS)
        def _(j):
            @pl.when(j < nz)
            def _():
                pltpu.async_copy(
                    zbuf, acc.at[pl.ds(zbase + j * ZROWS, ZROWS)], sem_z)

        @pl.loop(0, ZLAST // ZROWS)
        def _(j):
            @pl.when(j < nz)
            def _():
                pltpu.make_async_copy(
                    zbuf, acc.at[pl.ds(zbase + j * ZROWS, ZROWS)],
                    sem_z).wait()

        plsc.subcore_barrier()

        lanes = lax.iota(jnp.int32, 16)

        @pl.when(s == 0)
        def _():
            pltpu.sync_copy(x_hbm.at[pl.ds(r8v, 16)], sbuf)
            pltpu.sync_copy(keys_hbm.at[pl.ds(kb0 + r8v, 16)], skbuf)
            v = skbuf[...]
            valid = (v >= 0) & (v < HALF)
            skbuf[...] = jnp.where(valid, v, DUMMY + lanes)
            pltpu.sync_copy(sbuf, acc.at[skbuf], add=True)

        @pl.loop(0, (nfull + NBUF - 1) // NBUF)
        def _(g):
            for b in range(NBUF):
                i = NBUF * g + b

                @pl.when(i < nfull)
                def _(i=i, b=b):
                    wait_load(i, *bufs[b])
                    scatter(bufs[b][0], bufs[b][1])

                    @pl.when(i + NBUF < nfull)
                    def _():
                        start_load(i + NBUF, *bufs[b])

        @pl.when((s == NS - 1) & (rem > 0))
        def _():
            blockstart = pl.multiple_of(jnp.maximum(end_c - BLK, 0), 8)
            lo = base_s + (count_last // BLK) * BLK
            pltpu.sync_copy(x_hbm.at[pl.ds(blockstart, BLK)], xbuf_a)
            pltpu.sync_copy(keys_hbm.at[pl.ds(kb0 + blockstart, BLK)], kb_a)
            for q in range(BLK // 16):
                pos = blockstart + q * 16 + lanes
                v = kb_a[pl.ds(q * 16, 16)]
                valid = (pos >= lo) & (pos < end_c)
                kb_a[pl.ds(q * 16, 16)] = jnp.where(
                    valid, v, DUMMY + lanes)
            scatter(xbuf_a, kb_a)

        plsc.subcore_barrier()

        obase = pl.multiple_of(c * HALF + s * WSTRIPE, 8)

        @pl.when(s < NS - 1)
        def _():
            pltpu.sync_copy(
                acc.at[pl.ds(s * WSTRIPE, WSTRIPE)],
                out_hbm.at[pl.ds(obase, WSTRIPE)],
            )

        @pl.when(s == NS - 1)
        def _():
            pltpu.sync_copy(
                acc.at[pl.ds((NS - 1) * WSTRIPE, WLAST)],
                out_hbm.at[pl.ds(obase, WLAST)],
            )

    return k(X, keysl2, r8arr)


@jax.jit
def kernel(X, keys):
    keys32 = keys.astype(jnp.int32)
    r_star = jnp.searchsorted(keys32, jnp.int32(HALF)).astype(jnp.int32)
    r8 = jnp.minimum((r_star // 8) * 8, N_ROWS - 16)
    r8arr = jnp.full((16,), r8, jnp.int32)
    keysl2 = jnp.concatenate([keys32, keys32 - HALF])
    return _sc_segment_sum(X, keysl2, r8arr)
